# baseline (speedup 1.0000x reference)
import functools

import jax
import jax.numpy as jnp
from jax import lax
from jax.experimental import pallas as pl
from jax.experimental.pallas import tpu as pltpu
from jax.experimental.pallas import tpu_sc as plsc

_EMBED = 64
_NC = 2
_NS = 16
_NW = _NC * _NS
_BLK = 128
_L = 16


def _make_kernel(batch, hist):
    r_per_w = batch // _NW
    nrb = r_per_w // _BLK
    mesh = plsc.VectorSubcoreMesh(core_axis_name="c", subcore_axis_name="s")

    @functools.partial(
        pl.kernel,
        out_type=jax.ShapeDtypeStruct((hist, _EMBED, batch), jnp.float32),
        mesh=mesh,
        scratch_types=[
            pltpu.VMEM((2, r_per_w), jnp.int32),
            pltpu.VMEM((2, r_per_w), jnp.int32),
            pltpu.VMEM((2, r_per_w), jnp.int32),
            pltpu.VMEM((nrb, _BLK, 2 * _EMBED), jnp.float32),
            pltpu.VMEM((nrb, _EMBED, _BLK), jnp.float32),
            pltpu.SemaphoreType.DMA,
            pltpu.SemaphoreType.DMA,
            pltpu.SemaphoreType.DMA,
        ],
        compiler_params=pltpu.CompilerParams(
            needs_layout_passes=False, disable_bounds_checks=True),
    )
    def body(xt_hbm, tab_hbm, out_hbm, idx_v, pidx_v, hoff_v, rows_v, ot_v,
             isem, gsem, wsem):
        wid = lax.axis_index("s") * _NC + lax.axis_index("c")
        r0 = wid * r_per_w

        def fire_idx(h, p):
            pltpu.make_async_copy(
                xt_hbm.at[h, pl.ds(r0, r_per_w)], idx_v.at[p], isem).start()

        def wait_idx(h, p):
            pltpu.make_async_copy(
                xt_hbm.at[h, pl.ds(r0, r_per_w)], idx_v.at[p], isem).wait()

        def compute_pidx(p):
            for i in range(r_per_w // _L):
                v = idx_v[p, pl.ds(i * _L, _L)]
                pidx_v[p, pl.ds(i * _L, _L)] = lax.shift_right_logical(v, 1)
                hoff_v[p, pl.ds(i * _L, _L)] = lax.shift_left(
                    lax.bitwise_and(v, 1), 6)

        def fire_gather(p, rb):
            pltpu.make_async_copy(
                tab_hbm.at[pidx_v.at[p, pl.ds(rb * _BLK, _BLK)]],
                rows_v.at[rb], gsem).start()

        def wait_gather(p, rb):
            pltpu.make_async_copy(
                tab_hbm.at[pidx_v.at[p, pl.ds(rb * _BLK, _BLK)]],
                rows_v.at[rb], gsem).wait()

        def out_copy(h, rb):
            return pltpu.make_async_copy(
                ot_v.at[rb], out_hbm.at[h, :, pl.ds(r0 + rb * _BLK, _BLK)],
                wsem)

        rowids = [lax.iota(jnp.int32, _L) + g * _L for g in range(_BLK // _L)]

        def transpose_block(p, rb):
            hvec = [hoff_v[p, pl.ds(rb * _BLK + g * _L, _L)]
                    for g in range(_BLK // _L)]

            @pl.loop(0, _EMBED, step=8)
            def _e(e0):
                for de in range(8):
                    e = e0 + de
                    for g in range(_BLK // _L):
                        vals = plsc.load_gather(
                            rows_v.at[rb], [rowids[g], hvec[g] + e])
                        ot_v[rb, e, pl.ds(g * _L, _L)] = vals

        fire_idx(0, 0)
        wait_idx(0, 0)
        compute_pidx(0)
        fire_idx(1, 1)
        for rb in range(nrb):
            fire_gather(0, rb)

        @pl.loop(0, hist)
        def _h(h):
            p = lax.rem(h, 2)
            q = 1 - p

            @pl.when(h + 1 < hist)
            def _stage():
                wait_idx(h + 1, q)
                compute_pidx(q)

                @pl.when(h + 2 < hist)
                def _pf():
                    fire_idx(h + 2, p)

            for rb in range(nrb):
                wait_gather(p, rb)

                @pl.when(h > 0)
                def _drain():
                    out_copy(h, rb).wait()

                transpose_block(p, rb)
                out_copy(h, rb).start()

                @pl.when(h + 1 < hist)
                def _next():
                    fire_gather(q, rb)

        for rb in range(nrb):
            out_copy(hist - 1, rb).wait()

    return body


def kernel(x, vectors):
    b, h = x.shape
    xt = jnp.transpose(x)
    tab2 = jnp.reshape(vectors, (vectors.shape[0] // 2, 2 * _EMBED))
    out_t = _make_kernel(b, h)(xt, tab2)
    return jnp.transpose(out_t, (2, 0, 1))


# speedup vs baseline: 3.3319x; 3.3319x over previous
import functools

import jax
import jax.numpy as jnp
from jax import lax
from jax.experimental import pallas as pl
from jax.experimental.pallas import tpu as pltpu
from jax.experimental.pallas import tpu_sc as plsc

_EMBED = 64
_NC = 2
_NS = 16
_NW = _NC * _NS
_BLK = 128
_L = 16


def _make_kernel(batch, hist):
    r_per_w = batch // _NW
    nrb = r_per_w // _BLK
    mesh = plsc.VectorSubcoreMesh(core_axis_name="c", subcore_axis_name="s")

    @functools.partial(
        pl.kernel,
        out_type=jax.ShapeDtypeStruct((hist, _EMBED, batch), jnp.float32),
        mesh=mesh,
        scratch_types=[
            pltpu.VMEM((2, r_per_w), jnp.int32),
            pltpu.VMEM((2, r_per_w), jnp.int32),
            pltpu.VMEM((2, r_per_w), jnp.int32),
            pltpu.VMEM((nrb, _BLK, 2 * _EMBED), jnp.float32),
            pltpu.VMEM((nrb, _EMBED, _BLK), jnp.float32),
            pltpu.SemaphoreType.DMA,
            pltpu.SemaphoreType.DMA,
            pltpu.SemaphoreType.DMA,
        ],
        compiler_params=pltpu.CompilerParams(
            needs_layout_passes=False, disable_bounds_checks=True),
    )
    def body(xt_hbm, tab_hbm, out_hbm, idx_v, pidx_v, hoff_v, rows_v, ot_v,
             isem, gsem, wsem):
        wid = lax.axis_index("s") * _NC + lax.axis_index("c")
        r0 = wid * r_per_w

        def fire_idx(h, p):
            pltpu.make_async_copy(
                xt_hbm.at[h, pl.ds(r0, r_per_w)], idx_v.at[p], isem).start()

        def wait_idx(h, p):
            pltpu.make_async_copy(
                xt_hbm.at[h, pl.ds(r0, r_per_w)], idx_v.at[p], isem).wait()

        def compute_pidx(p):
            for i in range(r_per_w // _L):
                v = idx_v[p, pl.ds(i * _L, _L)]
                pidx_v[p, pl.ds(i * _L, _L)] = lax.shift_right_logical(v, 1)
                hoff_v[p, pl.ds(i * _L, _L)] = lax.shift_left(
                    lax.bitwise_and(v, 1), 6)

        def fire_gather(p, rb):
            pltpu.make_async_copy(
                tab_hbm.at[pidx_v.at[p, pl.ds(rb * _BLK, _BLK)]],
                rows_v.at[rb], gsem).start()

        def wait_gather(p, rb):
            pltpu.make_async_copy(
                tab_hbm.at[pidx_v.at[p, pl.ds(rb * _BLK, _BLK)]],
                rows_v.at[rb], gsem).wait()

        def out_copy(h, rb):
            return pltpu.make_async_copy(
                ot_v.at[rb], out_hbm.at[h, :, pl.ds(r0 + rb * _BLK, _BLK)],
                wsem)

        lanes = lax.iota(jnp.int32, _L)
        masks = {s: (lax.bitwise_and(lanes, s) == 0) for s in (1, 2, 4, 8)}
        pup = {s: lax.rem(lanes - s + _L, _L) for s in (1, 2, 4, 8)}
        pdn = {s: lax.rem(lanes + s, _L) for s in (1, 2, 4, 8)}

        def transpose_block(p, rb):
            @pl.loop(0, _BLK // _L)
            def _g(g):
                hvec = hoff_v[p, pl.ds(rb * _BLK + g * _L, _L)]
                hs = [hvec[k] for k in range(_L)]
                for eb in range(_EMBED // _L):
                    vs = [rows_v[rb, g * _L + k,
                                 pl.ds(hs[k] + eb * _L, _L)]
                          for k in range(_L)]
                    for s in (1, 2, 4, 8):
                        m, u, d = masks[s], pup[s], pdn[s]
                        for i in range(_L):
                            if i & s:
                                continue
                            a, b = vs[i], vs[i | s]
                            vs[i] = lax.select(m, a, b[u])
                            vs[i | s] = lax.select(m, a[d], b)
                    for k in range(_L):
                        ot_v[rb, eb * _L + k, pl.ds(g * _L, _L)] = vs[k]

        fire_idx(0, 0)
        wait_idx(0, 0)
        compute_pidx(0)
        fire_idx(1, 1)
        for rb in range(nrb):
            fire_gather(0, rb)

        @pl.loop(0, hist)
        def _h(h):
            p = lax.rem(h, 2)
            q = 1 - p

            @pl.when(h + 1 < hist)
            def _stage():
                wait_idx(h + 1, q)
                compute_pidx(q)

                @pl.when(h + 2 < hist)
                def _pf():
                    fire_idx(h + 2, p)

            for rb in range(nrb):
                wait_gather(p, rb)

                @pl.when(h > 0)
                def _drain():
                    out_copy(h, rb).wait()

                transpose_block(p, rb)
                out_copy(h, rb).start()

                @pl.when(h + 1 < hist)
                def _next():
                    fire_gather(q, rb)

        for rb in range(nrb):
            out_copy(hist - 1, rb).wait()

    return body


def kernel(x, vectors):
    b, h = x.shape
    xt = jnp.transpose(x)
    tab2 = jnp.reshape(vectors, (vectors.shape[0] // 2, 2 * _EMBED))
    out_t = _make_kernel(b, h)(xt, tab2)
    return jnp.transpose(out_t, (2, 0, 1))
